# baseline (speedup 1.0000x reference)
import functools

import jax
import jax.numpy as jnp
from jax import lax
from jax.experimental import pallas as pl
from jax.experimental.pallas import tpu as pltpu
from jax.experimental.pallas import tpu_sc as plsc

B = 128
NA, H, KA = 256, 4, 128
NM, KM = 512, 256
NR = 64
NP, KP = 256, 128
D = 256
ROWS_OUT = H * KA + KM + NR + KP

NW = 32
SPW = B // NW
CH = 128
NB = 2
SB = 8


def _rank_one(x_ref, y_ref, o_ref, n, off):
    ii = lax.broadcasted_iota(jnp.int32, (n, n), 0)
    jj = lax.broadcasted_iota(jnp.int32, (n, n), 1)
    tri = (ii < jj).astype(jnp.uint32)
    x = x_ref[...]
    y = y_ref[...]
    d8 = jnp.sqrt(x * x + y * y)
    k2 = lax.bitcast_convert_type(d8, jnp.uint32) * 2
    for s in range(SB):
        row = jnp.broadcast_to(k2[s:s + 1, :], (n, n))
        before = row.T < (row + tri)
        o_ref[s:s + 1, off:off + n] = jnp.sum(
            before.astype(jnp.int32), axis=0, keepdims=True)


def _rank_kernel(xa, ya, xm, ym, xp, yp, r_out):
    _rank_one(xa, ya, r_out, NA, 0)
    _rank_one(xm, ym, r_out, NM, NA)
    _rank_one(xp, yp, r_out, NP, NA + NM)


def _ranks(agent_poses, map_poses, polygon_poses):
    def split(p):
        return p[:, :, 0], p[:, :, 1]

    xa, ya = split(agent_poses)
    xm, ym = split(map_poses)
    xp, yp = split(polygon_poses)

    def row_spec(n):
        return pl.BlockSpec((SB, n), lambda b: (b, 0))

    nb = xa.shape[0]
    return pl.pallas_call(
        _rank_kernel,
        grid=(nb // SB,),
        in_specs=[row_spec(NA), row_spec(NA), row_spec(NM), row_spec(NM),
                  row_spec(NP), row_spec(NP)],
        out_specs=row_spec(NA + NM + NP),
        out_shape=jax.ShapeDtypeStruct((nb, NA + NM + NP), jnp.int32),
    )(xa, ya, xm, ym, xp, yp)


def _sc_body(off, spw, out, af, mf, rf, pf, rk, ranks_v, idx_v, buf, sem):
    wid = lax.axis_index("s") * 2 + lax.axis_index("c")

    def build_agent(i, c):
        b, slot = c
        r = ranks_v[pl.ds(slot + i * 16, 16)]
        iv = lax.iota(jnp.int32, 16) + i * 16
        m = r < KA
        for h in range(H):
            pos = r + (h * KA) + slot
            plsc.store_scatter(idx_v, [pos], iv + (b * 1024 + h * NA), mask=m)
        return c

    def build_map(i, c):
        b, slot = c
        r = ranks_v[pl.ds(slot + 256 + i * 16, 16)]
        iv = lax.iota(jnp.int32, 16) + i * 16
        m = r < KM
        plsc.store_scatter(idx_v, [r + 512 + slot], iv + b * NM, mask=m)
        return c

    def build_poly(i, c):
        b, slot = c
        r = ranks_v[pl.ds(slot + 768 + i * 16, 16)]
        iv = lax.iota(jnp.int32, 16) + i * 16
        m = r < KP
        plsc.store_scatter(idx_v, [r + 768 + slot], iv + b * NP, mask=m)
        return c

    plan = [(af, 0, 0, CH), (af, 1, 128, CH), (af, 2, 256, CH),
            (af, 3, 384, CH), (mf, 4, 512, CH), (mf, 5, 640, CH),
            (pf, 6, 832, CH), (None, 0, 512 + KM, NR)]
    TPS = len(plan)
    NT = spw * TPS
    bufs = [buf.at[i] for i in range(NB)]
    sin = [sem.at[i] for i in range(NB)]
    sout = [sem.at[NB + i] for i in range(NB)]
    pend_in = [None] * NB
    pend_out = [None] * NB
    pend_rank = [None, None]
    scene_l = [wid * spw + s for s in range(spw)]
    scene_b = [off + wid * spw + s for s in range(spw)]

    def start_rank(s):
        if s >= spw:
            return
        slot = (s % 2) * 1024
        pend_rank[s % 2] = pltpu.async_copy(
            rk.at[pl.ds(scene_l[s] * 1024, 1024)],
            ranks_v.at[pl.ds(slot, 1024)], sem.at[2 * NB + (s % 2)])

    def start_in(t, k):
        s = t // TPS
        b = scene_b[s]
        slot = (s % 2) * 1024
        if t % TPS == 0:
            pend_rank[s % 2].wait()
            start_rank(s + 1)
            lax.fori_loop(0, NA // 16, build_agent, (b, slot))
            lax.fori_loop(0, NM // 16, build_map, (b, slot))
            lax.fori_loop(0, NP // 16, build_poly, (b, slot))
        tbl, row, _, nrows = plan[t % TPS]
        if tbl is None:
            pend_in[k] = pltpu.async_copy(
                rf.at[pl.ds(b * NR, NR)], bufs[k].at[pl.ds(0, NR)], sin[k])
        else:
            pend_in[k] = pltpu.async_copy(
                tbl.at[idx_v.at[pl.ds(slot + row * CH, nrows)]],
                bufs[k].at[pl.ds(0, nrows)], sin[k])

    start_rank(0)
    for t in range(NB - 1):
        start_in(t, t)
    for t in range(NT):
        k = t % NB
        if t + NB - 1 < NT:
            k2 = (t + NB - 1) % NB
            if pend_out[k2] is not None:
                pend_out[k2].wait()
            start_in(t + NB - 1, k2)
        pend_in[k].wait()
        _, _, oo, nrows = plan[t % TPS]
        out_off = scene_b[t // TPS] * ROWS_OUT + oo
        pend_out[k] = pltpu.async_copy(
            bufs[k].at[pl.ds(0, nrows)], out.at[pl.ds(out_off, nrows)],
            sout[k])
    for k in range(NB):
        pend_out[k].wait()


@jax.jit
def kernel(agent_feats, agent_poses, map_feats, map_poses, route_feats,
           polygon_feats, polygon_poses):
    HB = B // 2
    af2 = agent_feats.reshape(B * H * NA, D)
    mf2 = map_feats.reshape(B * NM, D)
    rf2 = route_feats.reshape(B * NR, D)
    pf2 = polygon_feats.reshape(B * NP, D)

    out_ref = jax.new_ref(lax.empty((B * ROWS_OUT, D), jnp.float32))
    for half, off in enumerate((0, HB)):
        rk = _ranks(agent_poses[off:off + HB], map_poses[off:off + HB],
                    polygon_poses[off:off + HB])
        sc = pl.kernel(
            functools.partial(_sc_body, off, HB // NW),
            out_type=(),
            mesh=plsc.VectorSubcoreMesh(core_axis_name="c",
                                        subcore_axis_name="s"),
            compiler_params=pltpu.CompilerParams(needs_layout_passes=False),
            scratch_types=[
                pltpu.VMEM((2048,), jnp.int32),
                pltpu.VMEM((2048,), jnp.int32),
                pltpu.VMEM((NB, CH, D), jnp.float32),
                pltpu.SemaphoreType.DMA((2 * NB + 2,)),
            ],
        )
        sc(out_ref, af2, mf2, rf2, pf2, rk.reshape(HB * (NA + NM + NP)))
    return out_ref[...].reshape(B, ROWS_OUT, D)


# speedup vs baseline: 1.0040x; 1.0040x over previous
import functools

import jax
import jax.numpy as jnp
from jax import lax
from jax.experimental import pallas as pl
from jax.experimental.pallas import tpu as pltpu
from jax.experimental.pallas import tpu_sc as plsc

B = 128
NA, H, KA = 256, 4, 128
NM, KM = 512, 256
NR = 64
NP, KP = 256, 128
D = 256
ROWS_OUT = H * KA + KM + NR + KP

NW = 32
SPW = B // NW
CH = 128
NB = 2
SB = 8


def _rank_one(x_ref, y_ref, o_ref, n, off):
    ii = lax.broadcasted_iota(jnp.int32, (n, n), 0)
    jj = lax.broadcasted_iota(jnp.int32, (n, n), 1)
    tri = (ii < jj).astype(jnp.uint32)
    x = x_ref[...]
    y = y_ref[...]
    d8 = jnp.sqrt(x * x + y * y)
    k2 = lax.bitcast_convert_type(d8, jnp.uint32) * 2
    for s in range(SB):
        row = jnp.broadcast_to(k2[s:s + 1, :], (n, n))
        before = row.T < (row + tri)
        o_ref[s:s + 1, off:off + n] = jnp.sum(
            before.astype(jnp.int32), axis=0, keepdims=True)


def _rank_kernel(xa, ya, xm, ym, xp, yp, r_out):
    _rank_one(xa, ya, r_out, NA, 0)
    _rank_one(xm, ym, r_out, NM, NA)
    _rank_one(xp, yp, r_out, NP, NA + NM)


def _ranks(agent_poses, map_poses, polygon_poses):
    def split(p):
        return p[:, :, 0], p[:, :, 1]

    xa, ya = split(agent_poses)
    xm, ym = split(map_poses)
    xp, yp = split(polygon_poses)

    def row_spec(n):
        return pl.BlockSpec((SB, n), lambda b: (b, 0))

    nb = xa.shape[0]
    return pl.pallas_call(
        _rank_kernel,
        grid=(nb // SB,),
        in_specs=[row_spec(NA), row_spec(NA), row_spec(NM), row_spec(NM),
                  row_spec(NP), row_spec(NP)],
        out_specs=row_spec(NA + NM + NP),
        out_shape=jax.ShapeDtypeStruct((nb, NA + NM + NP), jnp.int32),
    )(xa, ya, xm, ym, xp, yp)


def _sc_body(off, spw, out, af, mf, rf, pf, rk, ranks_v, idx_v, buf, sem):
    wid = lax.axis_index("s") * 2 + lax.axis_index("c")

    def build_agent(i, c):
        b, slot = c
        r = ranks_v[pl.ds(slot + i * 16, 16)]
        iv = lax.iota(jnp.int32, 16) + i * 16
        m = r < KA
        for h in range(H):
            pos = r + (h * KA) + slot
            plsc.store_scatter(idx_v, [pos], iv + (b * 1024 + h * NA), mask=m)
        return c

    def build_map(i, c):
        b, slot = c
        r = ranks_v[pl.ds(slot + 256 + i * 16, 16)]
        iv = lax.iota(jnp.int32, 16) + i * 16
        m = r < KM
        plsc.store_scatter(idx_v, [r + 512 + slot], iv + b * NM, mask=m)
        return c

    def build_poly(i, c):
        b, slot = c
        r = ranks_v[pl.ds(slot + 768 + i * 16, 16)]
        iv = lax.iota(jnp.int32, 16) + i * 16
        m = r < KP
        plsc.store_scatter(idx_v, [r + 768 + slot], iv + b * NP, mask=m)
        return c

    plan = [(af, 0, 0, CH), (af, 1, 128, CH), (af, 2, 256, CH),
            (af, 3, 384, CH), (mf, 4, 512, CH), (mf, 5, 640, CH),
            (pf, 6, 832, CH), (None, 0, 512 + KM, NR)]
    TPS = len(plan)
    NT = spw * TPS
    bufs = [buf.at[i] for i in range(NB)]
    sin = [sem.at[i] for i in range(NB)]
    sout = [sem.at[NB + i] for i in range(NB)]
    pend_in = [None] * NB
    pend_out = [None] * NB
    pend_rank = [None, None]
    scene_l = [wid * spw + s for s in range(spw)]
    scene_b = [off + wid * spw + s for s in range(spw)]

    def start_rank(s):
        if s >= spw:
            return
        slot = (s % 2) * 1024
        pend_rank[s % 2] = pltpu.async_copy(
            rk.at[pl.ds(scene_l[s] * 1024, 1024)],
            ranks_v.at[pl.ds(slot, 1024)], sem.at[2 * NB + (s % 2)])

    def start_in(t, k):
        s = t // TPS
        b = scene_b[s]
        slot = (s % 2) * 1024
        if t % TPS == 0:
            pend_rank[s % 2].wait()
            start_rank(s + 1)
            lax.fori_loop(0, NA // 16, build_agent, (b, slot))
            lax.fori_loop(0, NM // 16, build_map, (b, slot))
            lax.fori_loop(0, NP // 16, build_poly, (b, slot))
        tbl, row, _, nrows = plan[t % TPS]
        if tbl is None:
            pend_in[k] = pltpu.async_copy(
                rf.at[pl.ds(b * NR, NR)], bufs[k].at[pl.ds(0, NR)], sin[k])
        else:
            pend_in[k] = pltpu.async_copy(
                tbl.at[idx_v.at[pl.ds(slot + row * CH, nrows)]],
                bufs[k].at[pl.ds(0, nrows)], sin[k])

    start_rank(0)
    for t in range(NB - 1):
        start_in(t, t)
    for t in range(NT):
        k = t % NB
        if t + NB - 1 < NT:
            k2 = (t + NB - 1) % NB
            if pend_out[k2] is not None:
                pend_out[k2].wait()
            start_in(t + NB - 1, k2)
        pend_in[k].wait()
        _, _, oo, nrows = plan[t % TPS]
        out_off = scene_b[t // TPS] * ROWS_OUT + oo
        pend_out[k] = pltpu.async_copy(
            bufs[k].at[pl.ds(0, nrows)], out.at[pl.ds(out_off, nrows)],
            sout[k])
    for k in range(NB):
        pend_out[k].wait()


@jax.jit
def kernel(agent_feats, agent_poses, map_feats, map_poses, route_feats,
           polygon_feats, polygon_poses):
    HB = B // 2
    af2 = agent_feats.reshape(B * H * NA, D)
    mf2 = map_feats.reshape(B * NM, D)
    rf2 = route_feats.reshape(B * NR, D)
    pf2 = polygon_feats.reshape(B * NP, D)

    out_ref = jax.new_ref(lax.empty((B * ROWS_OUT, D), jnp.float32))
    for half, off in enumerate((0, HB)):
        rk = _ranks(agent_poses[off:off + HB], map_poses[off:off + HB],
                    polygon_poses[off:off + HB])
        sc = pl.kernel(
            functools.partial(_sc_body, off, HB // NW),
            out_type=(),
            mesh=plsc.VectorSubcoreMesh(core_axis_name="c",
                                        subcore_axis_name="s"),
            compiler_params=pltpu.CompilerParams(needs_layout_passes=False),
            cost_estimate=pl.CostEstimate(
                flops=0, transcendentals=0,
                bytes_accessed=2 * HB * ROWS_OUT * D * 4),
            scratch_types=[
                pltpu.VMEM((2048,), jnp.int32),
                pltpu.VMEM((2048,), jnp.int32),
                pltpu.VMEM((NB, CH, D), jnp.float32),
                pltpu.SemaphoreType.DMA((2 * NB + 2,)),
            ],
        )
        sc(out_ref, af2, mf2, rf2, pf2, rk.reshape(HB * (NA + NM + NP)))
    return out_ref[...].reshape(B, ROWS_OUT, D)
